# baseline (speedup 1.0000x reference)
import functools

import jax
import jax.numpy as jnp
from jax import lax
from jax.experimental import pallas as pl
from jax.experimental.pallas import tpu as pltpu
from jax.experimental.pallas import tpu_sc as plsc

N = 1024
C = 200
V = 100000
TOTAL = N * C
NC, NS, L = 2, 16, 16
NW = NC * NS
PER_W = TOTAL // NW
ROWS_W = N // NW
NVREG = PER_W // L
CHUNK = 128
NCHUNK = PER_W // CHUNK
MAGIC = 10486
SHIFT = 21

_mesh = plsc.VectorSubcoreMesh(core_axis_name="c", subcore_axis_name="s")


@functools.partial(
    pl.kernel,
    out_type=jax.ShapeDtypeStruct((TOTAL,), jnp.float32),
    mesh=_mesh,
    compiler_params=pltpu.CompilerParams(skip_device_barrier=True),
    scratch_types=[
        pltpu.VMEM((PER_W,), jnp.int32),
        pltpu.VMEM((PER_W,), jnp.int32),
        pltpu.VMEM((PER_W,), jnp.float32),
    ] + [pltpu.SemaphoreType.DMA] * 2,
)
def _sc_gather(x_hbm, y_hbm, out_hbm, xv, iv, vv, *sems):
    wid = lax.axis_index("s") * NC + lax.axis_index("c")
    base = wid * PER_W
    pltpu.sync_copy(x_hbm.at[pl.ds(base, PER_W)], xv)

    lane = lax.iota(jnp.int32, L)
    nch = len(sems)
    vch = NVREG // nch
    ech = PER_W // nch

    for c in range(nch):

        @plsc.parallel_loop(c * vch, (c + 1) * vch, unroll=16)
        def idx_body(j):
            o = j * L
            xi = xv[pl.ds(o, L)]
            iv[pl.ds(o, L)] = (
                lax.shift_left(jnp.bitwise_and(xi, ~7), 10)
                + lax.shift_left(jnp.bitwise_and(xi, 7), 7)
                + (jnp.full((L,), (base + o) & 0x1C7F, jnp.int32) + lane))

        pltpu.async_copy(
            y_hbm.at[iv.at[pl.ds(c * ech, ech)]],
            vv.at[pl.ds(c * ech, ech)],
            sems[c],
        )

    for c in range(nch):
        pltpu.make_async_copy(
            y_hbm.at[pl.ds(0, ech)], vv.at[pl.ds(c * ech, ech)], sems[c]
        ).wait()

        @plsc.parallel_loop(c * vch, (c + 1) * vch, unroll=16)
        def add_body(j):
            o = j * L
            vv[pl.ds(o, L)] = (
                vv[pl.ds(o, L)] + xv[pl.ds(o, L)].astype(jnp.float32))

        pltpu.sync_copy(
            vv.at[pl.ds(c * ech, ech)],
            out_hbm.at[pl.ds(base + c * ech, ech)])


def _phys(a):
    r, c = a.shape
    return (
        a.T.reshape(c // 8, 8, r // 128, 128)
        .transpose(0, 2, 1, 3)
        .reshape(-1)
    )


def kernel(x, y):
    out = _sc_gather(_phys(x), _phys(y))
    return (
        out.reshape(C // 8, N // 128, 8, 128)
        .transpose(0, 2, 1, 3)
        .reshape(C, N)
        .T
    )


# speedup vs baseline: 1.0152x; 1.0152x over previous
import functools

import jax
import jax.numpy as jnp
from jax import lax
from jax.experimental import pallas as pl
from jax.experimental.pallas import tpu as pltpu
from jax.experimental.pallas import tpu_sc as plsc

N = 1024
C = 200
V = 100000
TOTAL = N * C
NC, NS, L = 2, 16, 16
NW = NC * NS
PER_W = TOTAL // NW
NVREG = PER_W // L

_mesh = plsc.VectorSubcoreMesh(core_axis_name="c", subcore_axis_name="s")


@functools.partial(
    pl.kernel,
    out_type=jax.ShapeDtypeStruct((TOTAL,), jnp.float32),
    mesh=_mesh,
    compiler_params=pltpu.CompilerParams(skip_device_barrier=True),
    scratch_types=[
        pltpu.VMEM((PER_W,), jnp.int32),
        pltpu.VMEM((PER_W,), jnp.int32),
        pltpu.VMEM((PER_W,), jnp.float32),
    ] + [pltpu.SemaphoreType.DMA] * 2,
)
def _sc_gather(x_hbm, y_hbm, out_hbm, xv, iv, vv, *sems):
    wid = lax.axis_index("s") * NC + lax.axis_index("c")
    base = wid * PER_W
    pltpu.sync_copy(x_hbm.at[pl.ds(base, PER_W)], xv)

    lane = lax.iota(jnp.int32, L)
    nch = len(sems)
    vch = NVREG // nch
    ech = PER_W // nch

    for c in range(nch):

        @plsc.parallel_loop(c * vch, (c + 1) * vch, unroll=8)
        def idx_body(j):
            o = j * L
            xi = xv[pl.ds(o, L)]
            iv[pl.ds(o, L)] = (
                lax.shift_left(jnp.bitwise_and(xi, ~7), 10)
                + lax.shift_left(jnp.bitwise_and(xi, 7), 7)
                + (jnp.full((L,), (base + o) & 0x1C7F, jnp.int32) + lane))

        pltpu.async_copy(
            y_hbm.at[iv.at[pl.ds(c * ech, ech)]],
            vv.at[pl.ds(c * ech, ech)],
            sems[c],
        )

    for c in range(nch):
        pltpu.make_async_copy(
            y_hbm.at[pl.ds(0, ech)], vv.at[pl.ds(c * ech, ech)], sems[c]
        ).wait()

        @plsc.parallel_loop(c * vch, (c + 1) * vch, unroll=8)
        def add_body(j):
            o = j * L
            vv[pl.ds(o, L)] = (
                vv[pl.ds(o, L)] + xv[pl.ds(o, L)].astype(jnp.float32))

        pltpu.sync_copy(
            vv.at[pl.ds(c * ech, ech)],
            out_hbm.at[pl.ds(base + c * ech, ech)])


def _phys(a):
    r, c = a.shape
    return (
        a.T.reshape(c // 8, 8, r // 128, 128)
        .transpose(0, 2, 1, 3)
        .reshape(-1)
    )


def kernel(x, y):
    out = _sc_gather(_phys(x), _phys(y))
    return (
        out.reshape(C // 8, N // 128, 8, 128)
        .transpose(0, 2, 1, 3)
        .reshape(C, N)
        .T
    )
